# baseline (speedup 1.0000x reference)
import functools

import jax
import jax.numpy as jnp
from jax import lax
from jax.experimental import pallas as pl
from jax.experimental.pallas import tpu as pltpu
from jax.experimental.pallas import tpu_sc as plsc

T = 4096
D = 2048
E = 8
K = 2
F = 512
FS = 512
RSF = 2.5

N = T * K
TM = 256
NP = N + E * TM
NT = NP // TM
EPAD = 128


def _silu_and_mul(x):
    g = x[:, :F]
    u = x[:, F:]
    return (g * jax.nn.sigmoid(g)) * u


RTB = 2048


def _router_body(xb_ref, gw_ref, out_ref):
    logits = jnp.dot(xb_ref[...], gw_ref[...], preferred_element_type=jnp.float32)
    lane = lax.broadcasted_iota(jnp.int32, (RTB, EPAD), 1)
    neg = jnp.float32(-1e30)
    s = jnp.where(lane < E, jax.nn.sigmoid(logits), neg)
    m1 = jnp.max(s, axis=1, keepdims=True)
    i1 = jnp.argmax(s, axis=1)[:, None]
    s2 = jnp.where(lane == i1, neg, s)
    m2 = jnp.max(s2, axis=1, keepdims=True)
    i2 = jnp.argmax(s2, axis=1)[:, None]
    denom = m1 + m2
    out_ref[...] = (jnp.where(lane == 0, i1.astype(jnp.float32), 0.0)
                    + jnp.where(lane == 1, i2.astype(jnp.float32), 0.0)
                    + jnp.where(lane == 2, m1 / denom, 0.0)
                    + jnp.where(lane == 3, m2 / denom, 0.0))


def _router(xb, gw):
    return pl.pallas_call(
        _router_body,
        grid=(T // RTB,),
        in_specs=[
            pl.BlockSpec((RTB, D), lambda b: (b, 0)),
            pl.BlockSpec((D, EPAD), lambda b: (0, 0)),
        ],
        out_specs=pl.BlockSpec((RTB, EPAD), lambda b: (b, 0)),
        out_shape=jax.ShapeDtypeStruct((T, EPAD), jnp.float32),
    )(xb, gw)


def _sc_gather(table, idx, ch):
    info = plsc.get_sparse_core_info()
    nw = info.num_cores * info.num_subcores
    b, d = idx.shape[0], table.shape[1]
    per_w = b // nw
    nch = per_w // ch
    mesh = plsc.VectorSubcoreMesh(core_axis_name="c", subcore_axis_name="s")

    @functools.partial(
        pl.kernel, mesh=mesh,
        out_type=jax.ShapeDtypeStruct((b, d), table.dtype),
        scratch_types=[
            pltpu.VMEM((per_w,), jnp.int32),
            pltpu.VMEM((ch, d), table.dtype),
            pltpu.VMEM((ch, d), table.dtype),
            pltpu.SemaphoreType.DMA,
            pltpu.SemaphoreType.DMA,
        ],
    )
    def k(table_hbm, idx_hbm, out_hbm, idx_v, rows0, rows1, sem0, sem1):
        wid = lax.axis_index("s") * info.num_cores + lax.axis_index("c")
        base = wid * per_w
        pltpu.sync_copy(idx_hbm.at[pl.ds(base, per_w)], idx_v)
        bufs = (rows0, rows1)
        sems = (sem0, sem1)

        def start(c, slot):
            pltpu.async_copy(table_hbm.at[idx_v.at[pl.ds(c * ch, ch)]],
                             bufs[slot], sems[slot])

        start(0, 0)
        for c in range(nch):
            slot = c % 2
            if c + 1 < nch:
                start(c + 1, 1 - slot)
            pltpu.make_async_copy(table_hbm.at[idx_v.at[pl.ds(c * ch, ch)]],
                                  bufs[slot], sems[slot]).wait()
            pltpu.sync_copy(bufs[slot], out_hbm.at[pl.ds(base + c * ch, ch)])

    return k(table, idx)


def _gmm_body(gid_ref, live_ref, xs_ref, swp_ref, wgu_ref, wd_ref, out_ref):
    i = pl.program_id(0)

    @pl.when(live_ref[i] == 1)
    def _():
        h = _silu_and_mul(jnp.dot(xs_ref[...], wgu_ref[0],
                                  preferred_element_type=jnp.float32))
        hd = jnp.dot(h.astype(jnp.bfloat16), wd_ref[0],
                     preferred_element_type=jnp.float32)
        out_ref[...] = (hd * swp_ref[...]).astype(jnp.bfloat16)


def _gmm(gid, live, xs, swp, wgu, wd):
    grid_spec = pltpu.PrefetchScalarGridSpec(
        num_scalar_prefetch=2,
        grid=(NT,),
        in_specs=[
            pl.BlockSpec((TM, D), lambda i, gid, live: (i, 0)),
            pl.BlockSpec((TM, 1), lambda i, gid, live: (i, 0)),
            pl.BlockSpec((1, D, 2 * F), lambda i, gid, live: (gid[i], 0, 0)),
            pl.BlockSpec((1, F, D), lambda i, gid, live: (gid[i], 0, 0)),
        ],
        out_specs=pl.BlockSpec((TM, D), lambda i, gid, live: (i, 0)),
    )
    return pl.pallas_call(
        _gmm_body,
        grid_spec=grid_spec,
        out_shape=jax.ShapeDtypeStruct((NP, D), jnp.bfloat16),
        compiler_params=pltpu.CompilerParams(
            dimension_semantics=("arbitrary",),
        ),
    )(gid, live, xs, swp, wgu, wd)


STB = 512


def _final_body(xb_ref, sgu_ref, sdn_ref, of_ref, out_ref):
    h = _silu_and_mul(jnp.dot(xb_ref[...], sgu_ref[...],
                              preferred_element_type=jnp.float32))
    shared = jnp.dot(h.astype(jnp.bfloat16), sdn_ref[...],
                     preferred_element_type=jnp.float32)
    routed = jnp.sum(of_ref[...].astype(jnp.float32).reshape(STB, K, D), axis=1)
    out_ref[...] = routed + shared


def _final(xb, sgu, sdn, of):
    return pl.pallas_call(
        _final_body,
        grid=(T // STB,),
        in_specs=[
            pl.BlockSpec((STB, D), lambda b: (b, 0)),
            pl.BlockSpec((D, 2 * FS), lambda b: (0, 0)),
            pl.BlockSpec((FS, D), lambda b: (0, 0)),
            pl.BlockSpec((K * STB, D), lambda b: (b, 0)),
        ],
        out_specs=pl.BlockSpec((STB, D), lambda b: (b, 0)),
        out_shape=jax.ShapeDtypeStruct((T, D), jnp.float32),
    )(xb, sgu, sdn, of)


@jax.jit
def kernel(hidden_states, gate_w, w_gate_up, w_down, shared_gate_up, shared_down):
    x = hidden_states
    xb = x.astype(jnp.bfloat16)
    gw = jnp.pad(gate_w, ((0, 0), (0, EPAD - E))).astype(jnp.bfloat16)
    wgu = w_gate_up.astype(jnp.bfloat16)
    wd = w_down.astype(jnp.bfloat16)
    sgu = shared_gate_up.astype(jnp.bfloat16)
    sdn = shared_down.astype(jnp.bfloat16)

    r = _router(xb, gw)
    e1 = r[:, 0].astype(jnp.int32)
    e2 = r[:, 1].astype(jnp.int32)
    w1 = r[:, 2]
    w2 = r[:, 3]

    ef = jnp.stack([e1, e2], axis=1).reshape(-1)
    wf = jnp.stack([w1, w2], axis=1).reshape(-1) * RSF
    oh = (ef[:, None] == jnp.arange(E)[None, :]).astype(jnp.int32)
    rank = jnp.take_along_axis(jnp.cumsum(oh, axis=0) - oh, ef[:, None], axis=1)[:, 0]
    counts = jnp.sum(oh, axis=0)
    p = ((counts + TM - 1) // TM) * TM
    ends = jnp.cumsum(p)
    pstart = ends - p
    pos = (pstart[ef] + rank).astype(jnp.int32)
    stp = jnp.zeros((NP,), jnp.int32).at[pos].set(
        (jnp.arange(N, dtype=jnp.int32) // K))
    swp = jnp.zeros((NP,), jnp.float32).at[pos].set(wf).reshape(NP, 1)
    tile_base = jnp.arange(NT, dtype=jnp.int32) * TM
    gid = jnp.clip(jnp.searchsorted(ends, tile_base, side="right"),
                   0, E - 1).astype(jnp.int32)
    live = (tile_base < ends[E - 1]).astype(jnp.int32)

    xi = lax.bitcast_convert_type(xb.reshape(T, D // 2, 2), jnp.int32)
    xsi = _sc_gather(xi, stp, 40)
    xs = lax.bitcast_convert_type(xsi, jnp.bfloat16).reshape(NP, D)

    osort = _gmm(gid, live, xs, swp, wgu, wd)

    oi = lax.bitcast_convert_type(osort.reshape(NP, D // 2, 2), jnp.int32)
    ofi = _sc_gather(oi, pos, 32)
    of = lax.bitcast_convert_type(ofi, jnp.bfloat16).reshape(N, D)
    return _final(xb, sgu, sdn, of)


# speedup vs baseline: 5.0173x; 5.0173x over previous
import functools

import jax
import jax.numpy as jnp
from jax import lax
from jax.experimental import pallas as pl
from jax.experimental.pallas import tpu as pltpu
from jax.experimental.pallas import tpu_sc as plsc

T = 4096
D = 2048
E = 8
K = 2
F = 512
FS = 512
RSF = 2.5

N = T * K
TM = 256
NP = N + E * TM
NT = NP // TM
EPAD = 128


def _silu_and_mul(x):
    g = x[:, :F]
    u = x[:, F:]
    return (g * jax.nn.sigmoid(g)) * u


def _pack_row(xb):
    half = xb.shape[1] // 2
    lo = lax.bitcast_convert_type(xb[:, :half], jnp.uint16).astype(jnp.uint32)
    hi = lax.bitcast_convert_type(xb[:, half:], jnp.uint16).astype(jnp.uint32)
    return lax.bitcast_convert_type(lo | (hi << 16), jnp.int32)


def _unpack_row(xi):
    u = lax.bitcast_convert_type(xi, jnp.uint32)
    lo = lax.bitcast_convert_type((u & 0xFFFF).astype(jnp.uint16), jnp.bfloat16)
    hi = lax.bitcast_convert_type((u >> 16).astype(jnp.uint16), jnp.bfloat16)
    return jnp.concatenate([lo, hi], axis=1)


RTB = 2048


def _router_body(xb_ref, gw_ref, out_ref, xi_ref):
    logits = jnp.dot(xb_ref[...], gw_ref[...], preferred_element_type=jnp.float32)
    lane = lax.broadcasted_iota(jnp.int32, (RTB, EPAD), 1)
    neg = jnp.float32(-1e30)
    s = jnp.where(lane < E, jax.nn.sigmoid(logits), neg)
    m1 = jnp.max(s, axis=1, keepdims=True)
    i1 = jnp.argmax(s, axis=1)[:, None]
    s2 = jnp.where(lane == i1, neg, s)
    m2 = jnp.max(s2, axis=1, keepdims=True)
    i2 = jnp.argmax(s2, axis=1)[:, None]
    denom = m1 + m2
    out_ref[...] = (jnp.where(lane == 0, i1.astype(jnp.float32), 0.0)
                    + jnp.where(lane == 1, i2.astype(jnp.float32), 0.0)
                    + jnp.where(lane == 2, m1 / denom, 0.0)
                    + jnp.where(lane == 3, m2 / denom, 0.0))
    xi_ref[...] = _pack_row(xb_ref[...])


def _router(xb, gw):
    return pl.pallas_call(
        _router_body,
        grid=(T // RTB,),
        in_specs=[
            pl.BlockSpec((RTB, D), lambda b: (b, 0)),
            pl.BlockSpec((D, EPAD), lambda b: (0, 0)),
        ],
        out_specs=[pl.BlockSpec((RTB, EPAD), lambda b: (b, 0)),
                   pl.BlockSpec((RTB, D // 2), lambda b: (b, 0))],
        out_shape=[jax.ShapeDtypeStruct((T, EPAD), jnp.float32),
                   jax.ShapeDtypeStruct((T, D // 2), jnp.int32)],
    )(xb, gw)


def _sc_gather(table, idx, ch):
    info = plsc.get_sparse_core_info()
    nw = info.num_cores * info.num_subcores
    b, d = idx.shape[0], table.shape[1]
    per_w = b // nw
    nch = per_w // ch
    mesh = plsc.VectorSubcoreMesh(core_axis_name="c", subcore_axis_name="s")

    @functools.partial(
        pl.kernel, mesh=mesh,
        out_type=jax.ShapeDtypeStruct((b, d), table.dtype),
        scratch_types=[
            pltpu.VMEM((per_w,), jnp.int32),
            pltpu.VMEM((ch, d), table.dtype),
            pltpu.VMEM((ch, d), table.dtype),
            pltpu.SemaphoreType.DMA,
            pltpu.SemaphoreType.DMA,
        ],
    )
    def k(table_hbm, idx_hbm, out_hbm, idx_v, rows0, rows1, sem0, sem1):
        wid = lax.axis_index("s") * info.num_cores + lax.axis_index("c")
        base = wid * per_w
        pltpu.sync_copy(idx_hbm.at[pl.ds(base, per_w)], idx_v)
        bufs = (rows0, rows1)
        sems = (sem0, sem1)

        def start(c, slot):
            pltpu.async_copy(table_hbm.at[idx_v.at[pl.ds(c * ch, ch)]],
                             bufs[slot], sems[slot])

        start(0, 0)
        for c in range(nch):
            slot = c % 2
            if c + 1 < nch:
                start(c + 1, 1 - slot)
            pltpu.make_async_copy(table_hbm.at[idx_v.at[pl.ds(c * ch, ch)]],
                                  bufs[slot], sems[slot]).wait()
            pltpu.sync_copy(bufs[slot], out_hbm.at[pl.ds(base + c * ch, ch)])

    return k(table, idx)


def _gmm_body(gid_ref, live_ref, xs_ref, wgu_ref, wd_ref, out_ref):
    i = pl.program_id(0)

    @pl.when(live_ref[i] == 1)
    def _():
        xs = _unpack_row(xs_ref[...])
        h = _silu_and_mul(jnp.dot(xs, wgu_ref[0],
                                  preferred_element_type=jnp.float32))
        o = jnp.dot(h.astype(jnp.bfloat16), wd_ref[0],
                    preferred_element_type=jnp.float32)
        out_ref[...] = _pack_row(o.astype(jnp.bfloat16))


def _gmm(gid, live, xs, wgu, wd):
    grid_spec = pltpu.PrefetchScalarGridSpec(
        num_scalar_prefetch=2,
        grid=(NT,),
        in_specs=[
            pl.BlockSpec((TM, D // 2), lambda i, gid, live: (i, 0)),
            pl.BlockSpec((1, D, 2 * F), lambda i, gid, live: (gid[i], 0, 0)),
            pl.BlockSpec((1, F, D), lambda i, gid, live: (gid[i], 0, 0)),
        ],
        out_specs=pl.BlockSpec((TM, D // 2), lambda i, gid, live: (i, 0)),
    )
    return pl.pallas_call(
        _gmm_body,
        grid_spec=grid_spec,
        out_shape=jax.ShapeDtypeStruct((NP, D // 2), jnp.int32),
        compiler_params=pltpu.CompilerParams(
            dimension_semantics=("arbitrary",),
        ),
    )(gid, live, xs, wgu, wd)


STB = 512


def _final_body(xb_ref, sgu_ref, sdn_ref, of_ref, r_ref, out_ref):
    h = _silu_and_mul(jnp.dot(xb_ref[...], sgu_ref[...],
                              preferred_element_type=jnp.float32))
    shared = jnp.dot(h.astype(jnp.bfloat16), sdn_ref[...],
                     preferred_element_type=jnp.float32)
    ofr = _unpack_row(of_ref[...]).astype(jnp.float32).reshape(STB, K, D)
    w1 = r_ref[:, 2:3] * RSF
    w2 = r_ref[:, 3:4] * RSF
    out_ref[...] = w1 * ofr[:, 0, :] + w2 * ofr[:, 1, :] + shared


def _final(xb, sgu, sdn, of, r):
    return pl.pallas_call(
        _final_body,
        grid=(T // STB,),
        in_specs=[
            pl.BlockSpec((STB, D), lambda b: (b, 0)),
            pl.BlockSpec((D, 2 * FS), lambda b: (0, 0)),
            pl.BlockSpec((FS, D), lambda b: (0, 0)),
            pl.BlockSpec((K * STB, D // 2), lambda b: (b, 0)),
            pl.BlockSpec((STB, EPAD), lambda b: (b, 0)),
        ],
        out_specs=pl.BlockSpec((STB, D), lambda b: (b, 0)),
        out_shape=jax.ShapeDtypeStruct((T, D), jnp.float32),
    )(xb, sgu, sdn, of, r)


@jax.jit
def kernel(hidden_states, gate_w, w_gate_up, w_down, shared_gate_up, shared_down):
    x = hidden_states
    xb = x.astype(jnp.bfloat16)
    gw = jnp.pad(gate_w, ((0, 0), (0, EPAD - E))).astype(jnp.bfloat16)
    wgu = w_gate_up.astype(jnp.bfloat16)
    wd = w_down.astype(jnp.bfloat16)
    sgu = shared_gate_up.astype(jnp.bfloat16)
    sdn = shared_down.astype(jnp.bfloat16)

    r, xi = _router(xb, gw)
    e1 = r[:, 0].astype(jnp.int32)
    e2 = r[:, 1].astype(jnp.int32)
    w1 = r[:, 2]
    w2 = r[:, 3]

    ef = jnp.stack([e1, e2], axis=1).reshape(-1)
    oh = (ef[:, None] == jnp.arange(E)[None, :]).astype(jnp.int32)
    rank = jnp.sum((jnp.cumsum(oh, axis=0) - oh) * oh, axis=1)
    counts = jnp.sum(oh, axis=0)
    p = ((counts + TM - 1) // TM) * TM
    ends = jnp.cumsum(p)
    pstart = ends - p
    pos = (jnp.sum(pstart[None, :] * oh, axis=1) + rank).astype(jnp.int32)
    stp = (jnp.arange(NP, dtype=jnp.int32) % T).at[pos].set(
        (jnp.arange(N, dtype=jnp.int32) // K))
    tile_base = jnp.arange(NT, dtype=jnp.int32) * TM
    gid = jnp.clip(jnp.sum((tile_base[:, None] >= ends[None, :]).astype(jnp.int32),
                           axis=1), 0, E - 1).astype(jnp.int32)
    live = (tile_base < ends[E - 1]).astype(jnp.int32)

    xsi = _sc_gather(xi, stp, 40)

    osort = _gmm(gid, live, xsi, wgu, wd)

    of = _sc_gather(osort, pos, 32)
    return _final(xb, sgu, sdn, of, r)
